# baseline (speedup 1.0000x reference)
import functools

import jax
import jax.numpy as jnp
from jax.experimental import pallas as pl
from jax.experimental.pallas import tpu as pltpu
from jax.experimental.pallas import tpu_sc as plsc

N = 100
NP = 104
C = 28
CP = 32
H = 64
IG = 16
SELF = 36
GWIN = 128


def _bdot(a, b):
    bf16 = jnp.bfloat16
    return jnp.dot(a.astype(bf16), b.astype(bf16),
                   preferred_element_type=jnp.float32)


def _b(a):
    return a.astype(jnp.bfloat16).astype(jnp.float32)


def _stage_a_body(si_ref, xo_ref, Ws_ref, bs_ref, Wo_ref, bo_ref, Wwo_ref,
                  Wws_ref, bw_ref, v_ref, Wa1_ref, Wa3_ref, ba_ref,
                  w_ref, pre_ref, w16_ref, lidx_ref, g16_ref):
    f32 = jnp.float32
    Bb = si_ref.shape[0]

    se = jax.nn.relu(_bdot(si_ref[...], Ws_ref[...]) + bs_ref[...])

    O3 = xo_ref[...]
    O2 = O3.reshape(Bb * NP, CP)
    OES2 = jax.nn.relu(_bdot(O2, Wo_ref[...]) + bo_ref[...])

    satt = _bdot(se, Wws_ref[...]) + bw_ref[...]
    satt2 = jnp.broadcast_to(satt[:, None, :], (Bb, NP, H)).reshape(Bb * NP, H)
    ATT = jnp.tanh(_bdot(OES2, Wwo_ref[...]) + satt2)
    scores = jnp.sum(_b(ATT).reshape(Bb, NP, H) * _b(v_ref[...]).reshape(1, 1, H),
                     axis=2)

    jidx = jax.lax.broadcasted_iota(jnp.int32, (Bb, NP), 1)
    scores = jnp.where(jidx < N, scores, -1e30)

    m = jnp.max(scores, axis=1, keepdims=True)
    e = jnp.exp(scores - m)
    w = e / jnp.sum(e, axis=1, keepdims=True)
    w_ref[...] = w[:, :N]

    wi = w[:, :, None]
    wj = w[:, None, :]
    ii = jax.lax.broadcasted_iota(jnp.int32, (Bb, NP, NP), 1)
    jj = jax.lax.broadcasted_iota(jnp.int32, (Bb, NP, NP), 2)
    beats = (wi > wj) | ((wi == wj) & (ii < jj))
    wins3 = jnp.sum(beats.astype(f32), axis=2, keepdims=True)
    topf3 = jnp.where(wins3 >= float(NP - IG), 1.0, 0.0)

    OES3 = _b(OES2).reshape(Bb, NP, H)
    uw3 = _b(wi * (1.0 - topf3))
    u_e = jnp.sum(uw3 * OES3, axis=1)

    rank_l = jnp.sum(beats.astype(f32), axis=1)
    wcols, icols = [], []
    for k in range(IG):
        mk = rank_l == float(k)
        wcols.append(jnp.sum(jnp.where(mk, w, 0.0), axis=1, keepdims=True))
        icols.append(jnp.sum(jnp.where(mk, jidx, 0), axis=1, keepdims=True))
    w16_ref[...] = jnp.concatenate(wcols, axis=1)
    lidx = jnp.concatenate(icols, axis=1)
    lidx_ref[...] = lidx
    bio = jax.lax.broadcasted_iota(jnp.int32, (Bb, IG), 0)
    absb = pl.program_id(0) * Bb + bio
    g16_ref[...] = absb * (NP * CP // 128) + lidx // 4

    pre_ref[...] = (_bdot(se, Wa1_ref[...]) + _bdot(u_e, Wa3_ref[...])
                    + ba_ref[...])


def _stage_c_body(X_ref, w16_ref, lidx_ref, pre_ref, G_ref, R2_ref, Wig_ref,
                  big_ref, Wa2_ref, obs_ref):
    f32 = jnp.float32
    Bb = w16_ref.shape[0]
    lrep = jnp.dot(lidx_ref[...].astype(f32), G_ref[...],
                   preferred_element_type=f32)
    wrep = jnp.dot(w16_ref[...], G_ref[...], preferred_element_type=f32)
    qq = jax.lax.broadcasted_iota(jnp.int32, lrep.shape, 1) % 4
    hw = jnp.where((lrep.astype(jnp.int32) % 4) == qq, wrep, 0.0)
    mrep = jnp.dot(hw, R2_ref[...], preferred_element_type=f32)
    Xw = X_ref[...] * mrep
    i_e = jax.nn.relu(_bdot(Xw, Wig_ref[...]) + big_ref[...])
    obs_ref[...] = pre_ref[...] + _bdot(i_e, Wa2_ref[...])


def _sc_gather(xflat, gidx):
    nidx = gidx.shape[1]
    mesh = plsc.VectorSubcoreMesh(core_axis_name="core",
                                  subcore_axis_name="subcore")

    @functools.partial(
        pl.kernel,
        out_type=jax.ShapeDtypeStruct((nidx, 128), jnp.float32),
        mesh=mesh)
    def gather_kernel(x_hbm, i_hbm, o_hbm):
        def body(i_vmem, o_vmem):
            pltpu.sync_copy(x_hbm.at[i_vmem.at[0]], o_vmem)

        pltpu.emit_pipeline(
            body,
            grid=(nidx // GWIN,),
            in_specs=[pl.BlockSpec((1, GWIN), index_map=lambda i: (0, i))],
            out_specs=[pl.BlockSpec((GWIN, 128), index_map=lambda i: (i, 0))],
            core_axis_name=("core", "subcore"),
            dimension_semantics=(pltpu.PARALLEL,),
        )(i_hbm, o_hbm)

    return gather_kernel(xflat, gidx)


def _stage_a(si, xo, Ws, bs, Wo_p, bo, Wwo, Wws, bw, v, Wa1, Wa3, ba,
             Bb, interpret=False):
    B = si.shape[0]
    f32 = jnp.float32
    row = lambda a: a.reshape(1, -1)
    full = lambda shape: pl.BlockSpec(shape, lambda i: (0,) * len(shape))
    return pl.pallas_call(
        _stage_a_body,
        grid=(B // Bb,),
        in_specs=[
            pl.BlockSpec((Bb, SELF), lambda i: (i, 0)),
            pl.BlockSpec((Bb, NP, CP), lambda i: (i, 0, 0)),
            full((SELF, H)), full((1, H)),
            full((CP, H)), full((1, H)),
            full((H, H)), full((H, H)),
            full((1, H)), full((1, H)),
            full((H, H)), full((H, H)), full((1, H)),
        ],
        out_specs=[
            pl.BlockSpec((Bb, N), lambda i: (i, 0)),
            pl.BlockSpec((Bb, H), lambda i: (i, 0)),
            pl.BlockSpec((Bb, IG), lambda i: (i, 0)),
            pl.BlockSpec((Bb, IG), lambda i: (i, 0)),
            pl.BlockSpec((Bb, IG), lambda i: (i, 0)),
        ],
        out_shape=[
            jax.ShapeDtypeStruct((B, N), f32),
            jax.ShapeDtypeStruct((B, H), f32),
            jax.ShapeDtypeStruct((B, IG), f32),
            jax.ShapeDtypeStruct((B, IG), jnp.int32),
            jax.ShapeDtypeStruct((B, IG), jnp.int32),
        ],
        interpret=interpret,
    )(si, xo, Ws, row(bs), Wo_p, row(bo), Wwo, Wws, row(bw), row(v),
      Wa1, Wa3, row(ba))


def _stage_c(X2, w16, lidx, pre, G, R2, Wig2048, big, Wa2, Bb,
             interpret=False):
    B = w16.shape[0]
    f32 = jnp.float32
    row = lambda a: a.reshape(1, -1)
    full = lambda shape: pl.BlockSpec(shape, lambda i: (0,) * len(shape))
    return pl.pallas_call(
        _stage_c_body,
        grid=(B // Bb,),
        in_specs=[
            pl.BlockSpec((Bb, IG * 128), lambda i: (i, 0)),
            pl.BlockSpec((Bb, IG), lambda i: (i, 0)),
            pl.BlockSpec((Bb, IG), lambda i: (i, 0)),
            pl.BlockSpec((Bb, H), lambda i: (i, 0)),
            full((IG, 4 * IG)),
            full((4 * IG, IG * 128)),
            full((IG * 128, H)), full((1, H)),
            full((H, H)),
        ],
        out_specs=pl.BlockSpec((Bb, H), lambda i: (i, 0)),
        out_shape=jax.ShapeDtypeStruct((B, H), f32),
        interpret=interpret,
    )(X2, w16, lidx, pre, G, R2, Wig2048, row(big), Wa2)


@jax.jit
def kernel(x, Ws, bs, Wo, bo, Ww, bw, v, Wig, big, Wa, ba):
    B = x.shape[0]
    f32 = jnp.float32

    si = x[:, :SELF]
    xo = jnp.pad(x[:, SELF:].reshape(B, N, C),
                 ((0, 0), (0, NP - N), (0, CP - C)))
    Wo_p = jnp.pad(Wo, ((0, CP - C), (0, 0)))
    Wig_p = jnp.pad(Wig.reshape(IG, C, H),
                    ((0, 0), (0, CP - C), (0, 0)))
    Wig2048 = jnp.broadcast_to(Wig_p[:, None], (IG, 4, CP, H)).reshape(
        IG * 4 * CP, H)
    Wwo, Wws = Ww[:H], Ww[H:]
    Wa1, Wa2, Wa3 = Wa[:H], Wa[H:2 * H], Wa[2 * H:]
    G = jnp.kron(jnp.eye(IG, dtype=f32), jnp.ones((1, 4), f32))
    R2 = jnp.kron(jnp.eye(4 * IG, dtype=f32), jnp.ones((1, CP), f32))

    w, pre, w16, lidx, g16 = _stage_a(si, xo, Ws, bs, Wo_p, bo, Wwo, Wws, bw,
                                      v, Wa1, Wa3, ba, Bb=128)
    X = _sc_gather(xo.reshape(B * NP * CP // 128, 128),
                   g16.reshape(1, B * IG))
    obs = _stage_c(X.reshape(B, IG * 128), w16, lidx, pre, G, R2, Wig2048,
                   big, Wa2, Bb=512)
    return obs, w


# speedup vs baseline: 1.1633x; 1.1633x over previous
import functools

import jax
import jax.numpy as jnp
from jax.experimental import pallas as pl
from jax.experimental.pallas import tpu as pltpu
from jax.experimental.pallas import tpu_sc as plsc

N = 100
NP = 104
C = 28
CP = 32
H = 64
IG = 16
SELF = 36
GWIN = 128


def _bdot(a, b):
    bf16 = jnp.bfloat16
    return jnp.dot(a.astype(bf16), b.astype(bf16),
                   preferred_element_type=jnp.float32)


def _b(a):
    return a.astype(jnp.bfloat16).astype(jnp.float32)


def _stage_a_body(si_ref, xo_ref, Ws_ref, bs_ref, Wo_ref, bo_ref, Wwo_ref,
                  Wws_ref, bw_ref, v_ref, Wa1_ref, Wa3_ref, ba_ref,
                  w_ref, pre_ref, w16_ref, lidx_ref, g16_ref):
    f32 = jnp.float32
    Bb = si_ref.shape[0]

    se = jax.nn.relu(_bdot(si_ref[...], Ws_ref[...]) + bs_ref[...])

    O3 = xo_ref[...]
    O2 = O3.reshape(Bb * NP, CP)
    OES2 = jax.nn.relu(_bdot(O2, Wo_ref[...]) + bo_ref[...])

    satt = _bdot(se, Wws_ref[...]) + bw_ref[...]
    satt2 = jnp.broadcast_to(satt[:, None, :], (Bb, NP, H)).reshape(Bb * NP, H)
    ATT = jnp.tanh(_bdot(OES2, Wwo_ref[...]) + satt2)
    scores = jnp.sum(_b(ATT).reshape(Bb, NP, H) * _b(v_ref[...]).reshape(1, 1, H),
                     axis=2)

    jidx = jax.lax.broadcasted_iota(jnp.int32, (Bb, NP), 1)
    scores = jnp.where(jidx < N, scores, -1e30)

    m = jnp.max(scores, axis=1, keepdims=True)
    e = jnp.exp(scores - m)
    w = e / jnp.sum(e, axis=1, keepdims=True)
    w_ref[...] = w[:, :N]

    jidxf = jidx.astype(f32)
    mcur = w
    wcols, icols = [], []
    for k in range(IG):
        mx = jnp.max(mcur, axis=1, keepdims=True)
        hit = mcur == mx
        idxk = jnp.min(jnp.where(hit, jidxf, float(NP)), axis=1,
                       keepdims=True)
        wcols.append(mx)
        icols.append(idxk)
        mcur = jnp.where(jidxf == idxk, -1.0, mcur)
    w16_ref[...] = jnp.concatenate(wcols, axis=1)
    lidx = jnp.concatenate(icols, axis=1).astype(jnp.int32)
    lidx_ref[...] = lidx

    OES3 = _b(OES2).reshape(Bb, NP, H)
    uw3 = _b(jnp.maximum(mcur, 0.0))[:, :, None]
    u_e = jnp.sum(uw3 * OES3, axis=1)
    bio = jax.lax.broadcasted_iota(jnp.int32, (Bb, IG), 0)
    absb = pl.program_id(0) * Bb + bio
    g16_ref[...] = absb * (NP * CP // 128) + lidx // 4

    pre_ref[...] = (_bdot(se, Wa1_ref[...]) + _bdot(u_e, Wa3_ref[...])
                    + ba_ref[...])


def _stage_c_body(X_ref, w16_ref, lidx_ref, pre_ref, G_ref, R2_ref, Wig_ref,
                  big_ref, Wa2_ref, obs_ref):
    f32 = jnp.float32
    Bb = w16_ref.shape[0]
    lrep = jnp.dot(lidx_ref[...].astype(f32), G_ref[...],
                   preferred_element_type=f32)
    wrep = jnp.dot(w16_ref[...], G_ref[...], preferred_element_type=f32)
    qq = jax.lax.broadcasted_iota(jnp.int32, lrep.shape, 1) % 4
    hw = jnp.where((lrep.astype(jnp.int32) % 4) == qq, wrep, 0.0)
    mrep = jnp.dot(hw, R2_ref[...], preferred_element_type=f32)
    Xw = X_ref[...] * mrep
    i_e = jax.nn.relu(_bdot(Xw, Wig_ref[...]) + big_ref[...])
    obs_ref[...] = pre_ref[...] + _bdot(i_e, Wa2_ref[...])


def _sc_gather(xflat, gidx):
    nidx = gidx.shape[1]
    mesh = plsc.VectorSubcoreMesh(core_axis_name="core",
                                  subcore_axis_name="subcore")

    @functools.partial(
        pl.kernel,
        out_type=jax.ShapeDtypeStruct((nidx, 128), jnp.float32),
        mesh=mesh)
    def gather_kernel(x_hbm, i_hbm, o_hbm):
        def body(i_vmem, o_vmem):
            pltpu.sync_copy(x_hbm.at[i_vmem.at[0]], o_vmem)

        pltpu.emit_pipeline(
            body,
            grid=(nidx // GWIN,),
            in_specs=[pl.BlockSpec((1, GWIN), index_map=lambda i: (0, i))],
            out_specs=[pl.BlockSpec((GWIN, 128), index_map=lambda i: (i, 0))],
            core_axis_name=("core", "subcore"),
            dimension_semantics=(pltpu.PARALLEL,),
        )(i_hbm, o_hbm)

    return gather_kernel(xflat, gidx)


def _stage_a(si, xo, Ws, bs, Wo_p, bo, Wwo, Wws, bw, v, Wa1, Wa3, ba,
             Bb, interpret=False):
    B = si.shape[0]
    f32 = jnp.float32
    row = lambda a: a.reshape(1, -1)
    full = lambda shape: pl.BlockSpec(shape, lambda i: (0,) * len(shape))
    return pl.pallas_call(
        _stage_a_body,
        grid=(B // Bb,),
        in_specs=[
            pl.BlockSpec((Bb, SELF), lambda i: (i, 0)),
            pl.BlockSpec((Bb, NP, CP), lambda i: (i, 0, 0)),
            full((SELF, H)), full((1, H)),
            full((CP, H)), full((1, H)),
            full((H, H)), full((H, H)),
            full((1, H)), full((1, H)),
            full((H, H)), full((H, H)), full((1, H)),
        ],
        out_specs=[
            pl.BlockSpec((Bb, N), lambda i: (i, 0)),
            pl.BlockSpec((Bb, H), lambda i: (i, 0)),
            pl.BlockSpec((Bb, IG), lambda i: (i, 0)),
            pl.BlockSpec((Bb, IG), lambda i: (i, 0)),
            pl.BlockSpec((Bb, IG), lambda i: (i, 0)),
        ],
        out_shape=[
            jax.ShapeDtypeStruct((B, N), f32),
            jax.ShapeDtypeStruct((B, H), f32),
            jax.ShapeDtypeStruct((B, IG), f32),
            jax.ShapeDtypeStruct((B, IG), jnp.int32),
            jax.ShapeDtypeStruct((B, IG), jnp.int32),
        ],
        interpret=interpret,
    )(si, xo, Ws, row(bs), Wo_p, row(bo), Wwo, Wws, row(bw), row(v),
      Wa1, Wa3, row(ba))


def _stage_c(X2, w16, lidx, pre, G, R2, Wig2048, big, Wa2, Bb,
             interpret=False):
    B = w16.shape[0]
    f32 = jnp.float32
    row = lambda a: a.reshape(1, -1)
    full = lambda shape: pl.BlockSpec(shape, lambda i: (0,) * len(shape))
    return pl.pallas_call(
        _stage_c_body,
        grid=(B // Bb,),
        in_specs=[
            pl.BlockSpec((Bb, IG * 128), lambda i: (i, 0)),
            pl.BlockSpec((Bb, IG), lambda i: (i, 0)),
            pl.BlockSpec((Bb, IG), lambda i: (i, 0)),
            pl.BlockSpec((Bb, H), lambda i: (i, 0)),
            full((IG, 4 * IG)),
            full((4 * IG, IG * 128)),
            full((IG * 128, H)), full((1, H)),
            full((H, H)),
        ],
        out_specs=pl.BlockSpec((Bb, H), lambda i: (i, 0)),
        out_shape=jax.ShapeDtypeStruct((B, H), f32),
        interpret=interpret,
    )(X2, w16, lidx, pre, G, R2, Wig2048, row(big), Wa2)


@jax.jit
def kernel(x, Ws, bs, Wo, bo, Ww, bw, v, Wig, big, Wa, ba):
    B = x.shape[0]
    f32 = jnp.float32

    si = x[:, :SELF]
    xo = jnp.pad(x[:, SELF:].reshape(B, N, C),
                 ((0, 0), (0, NP - N), (0, CP - C)))
    Wo_p = jnp.pad(Wo, ((0, CP - C), (0, 0)))
    Wig_p = jnp.pad(Wig.reshape(IG, C, H),
                    ((0, 0), (0, CP - C), (0, 0)))
    Wig2048 = jnp.broadcast_to(Wig_p[:, None], (IG, 4, CP, H)).reshape(
        IG * 4 * CP, H)
    Wwo, Wws = Ww[:H], Ww[H:]
    Wa1, Wa2, Wa3 = Wa[:H], Wa[H:2 * H], Wa[2 * H:]
    G = jnp.kron(jnp.eye(IG, dtype=f32), jnp.ones((1, 4), f32))
    R2 = jnp.kron(jnp.eye(4 * IG, dtype=f32), jnp.ones((1, CP), f32))

    w, pre, w16, lidx, g16 = _stage_a(si, xo, Ws, bs, Wo_p, bo, Wwo, Wws, bw,
                                      v, Wa1, Wa3, ba, Bb=128)
    X = _sc_gather(xo.reshape(B * NP * CP // 128, 128),
                   g16.reshape(1, B * IG))
    obs = _stage_c(X.reshape(B, IG * 128), w16, lidx, pre, G, R2, Wig2048,
                   big, Wa2, Bb=512)
    return obs, w


# speedup vs baseline: 1.1637x; 1.0003x over previous
import functools

import jax
import jax.numpy as jnp
from jax.experimental import pallas as pl
from jax.experimental.pallas import tpu as pltpu
from jax.experimental.pallas import tpu_sc as plsc

N = 100
NP = 104
C = 28
CP = 32
H = 64
IG = 16
SELF = 36
GWIN = 128


def _bdot(a, b):
    bf16 = jnp.bfloat16
    return jnp.dot(a.astype(bf16), b.astype(bf16),
                   preferred_element_type=jnp.float32)


def _b(a):
    return a.astype(jnp.bfloat16).astype(jnp.float32)


def _stage_a_body(si_ref, xo_ref, Ws_ref, bs_ref, Wo_ref, bo_ref, Wwo_ref,
                  Wws_ref, bw_ref, v_ref, Wa1_ref, Wa3_ref, ba_ref,
                  w_ref, pre_ref, w16_ref, lidx_ref, g16_ref):
    f32 = jnp.float32
    Bb = si_ref.shape[0]

    se = jax.nn.relu(_bdot(si_ref[...], Ws_ref[...]) + bs_ref[...])

    O3 = xo_ref[...]
    O2 = O3.reshape(Bb * NP, CP)
    OES2 = jax.nn.relu(_bdot(O2, Wo_ref[...]) + bo_ref[...])

    satt = _bdot(se, Wws_ref[...]) + bw_ref[...]
    satt2 = jnp.broadcast_to(satt[:, None, :], (Bb, NP, H)).reshape(Bb * NP, H)
    ATT = jnp.tanh(_bdot(OES2, Wwo_ref[...]) + satt2)
    scores = jnp.sum(_b(ATT).reshape(Bb, NP, H) * _b(v_ref[...]).reshape(1, 1, H),
                     axis=2)

    jidx = jax.lax.broadcasted_iota(jnp.int32, (Bb, NP), 1)
    scores = jnp.where(jidx < N, scores, -1e30)

    m = jnp.max(scores, axis=1, keepdims=True)
    e = jnp.exp(scores - m)
    w = e / jnp.sum(e, axis=1, keepdims=True)
    w_ref[...] = w[:, :N]

    jidxf = jidx.astype(f32)
    mcur = w
    wcols, icols = [], []
    for k in range(IG):
        mx = jnp.max(mcur, axis=1, keepdims=True)
        hit = mcur == mx
        idxk = jnp.min(jnp.where(hit, jidxf, float(NP)), axis=1,
                       keepdims=True)
        wcols.append(mx)
        icols.append(idxk)
        mcur = jnp.where(jidxf == idxk, -1.0, mcur)
    w16_ref[...] = jnp.concatenate(wcols, axis=1)
    lidx = jnp.concatenate(icols, axis=1).astype(jnp.int32)
    lidx_ref[...] = lidx

    OES3 = _b(OES2).reshape(Bb, NP, H)
    uw3 = _b(jnp.maximum(mcur, 0.0))[:, :, None]
    u_e = jnp.sum(uw3 * OES3, axis=1)
    bio = jax.lax.broadcasted_iota(jnp.int32, (Bb, IG), 0)
    absb = pl.program_id(0) * Bb + bio
    g16_ref[...] = absb * (NP * CP // 128) + lidx // 4

    pre_ref[...] = (_bdot(se, Wa1_ref[...]) + _bdot(u_e, Wa3_ref[...])
                    + ba_ref[...])


def _stage_c_body(X_ref, w16_ref, lidx_ref, pre_ref, G_ref, R2_ref, Wig_ref,
                  big_ref, Wa2_ref, obs_ref):
    f32 = jnp.float32
    Bb = w16_ref.shape[0]
    lrep = jnp.dot(lidx_ref[...].astype(f32), G_ref[...],
                   preferred_element_type=f32)
    wrep = jnp.dot(w16_ref[...], G_ref[...], preferred_element_type=f32)
    qq = jax.lax.broadcasted_iota(jnp.int32, lrep.shape, 1) % 4
    hw = jnp.where((lrep.astype(jnp.int32) % 4) == qq, wrep, 0.0)
    mrep = jnp.dot(hw, R2_ref[...], preferred_element_type=f32)
    Xw = X_ref[...] * mrep
    i_e = jax.nn.relu(_bdot(Xw, Wig_ref[...]) + big_ref[...])
    obs_ref[...] = pre_ref[...] + _bdot(i_e, Wa2_ref[...])


def _sc_gather(xflat, gidx):
    nidx = gidx.shape[1]
    mesh = plsc.VectorSubcoreMesh(core_axis_name="core",
                                  subcore_axis_name="subcore")

    @functools.partial(
        pl.kernel,
        out_type=jax.ShapeDtypeStruct((nidx, 128), jnp.float32),
        mesh=mesh)
    def gather_kernel(x_hbm, i_hbm, o_hbm):
        def body(i_vmem, o_vmem):
            pltpu.sync_copy(x_hbm.at[i_vmem.at[0]], o_vmem)

        pltpu.emit_pipeline(
            body,
            grid=(nidx // GWIN,),
            in_specs=[pl.BlockSpec((1, GWIN), index_map=lambda i: (0, i))],
            out_specs=[pl.BlockSpec((GWIN, 128), index_map=lambda i: (i, 0))],
            core_axis_name=("core", "subcore"),
            dimension_semantics=(pltpu.PARALLEL,),
        )(i_hbm, o_hbm)

    return gather_kernel(xflat, gidx)


def _stage_a(si, xo, Ws, bs, Wo_p, bo, Wwo, Wws, bw, v, Wa1, Wa3, ba,
             Bb, interpret=False):
    B = si.shape[0]
    f32 = jnp.float32
    row = lambda a: a.reshape(1, -1)
    full = lambda shape: pl.BlockSpec(shape, lambda i: (0,) * len(shape))
    return pl.pallas_call(
        _stage_a_body,
        grid=(B // Bb,),
        in_specs=[
            pl.BlockSpec((Bb, SELF), lambda i: (i, 0)),
            pl.BlockSpec((Bb, NP, CP), lambda i: (i, 0, 0)),
            full((SELF, H)), full((1, H)),
            full((CP, H)), full((1, H)),
            full((H, H)), full((H, H)),
            full((1, H)), full((1, H)),
            full((H, H)), full((H, H)), full((1, H)),
        ],
        out_specs=[
            pl.BlockSpec((Bb, N), lambda i: (i, 0)),
            pl.BlockSpec((Bb, H), lambda i: (i, 0)),
            pl.BlockSpec((Bb, IG), lambda i: (i, 0)),
            pl.BlockSpec((Bb, IG), lambda i: (i, 0)),
            pl.BlockSpec((Bb, IG), lambda i: (i, 0)),
        ],
        out_shape=[
            jax.ShapeDtypeStruct((B, N), f32),
            jax.ShapeDtypeStruct((B, H), f32),
            jax.ShapeDtypeStruct((B, IG), f32),
            jax.ShapeDtypeStruct((B, IG), jnp.int32),
            jax.ShapeDtypeStruct((B, IG), jnp.int32),
        ],
        compiler_params=pltpu.CompilerParams(
            dimension_semantics=("parallel",)),
        interpret=interpret,
    )(si, xo, Ws, row(bs), Wo_p, row(bo), Wwo, Wws, row(bw), row(v),
      Wa1, Wa3, row(ba))


def _stage_c(X2, w16, lidx, pre, G, R2, Wig2048, big, Wa2, Bb,
             interpret=False):
    B = w16.shape[0]
    f32 = jnp.float32
    row = lambda a: a.reshape(1, -1)
    full = lambda shape: pl.BlockSpec(shape, lambda i: (0,) * len(shape))
    return pl.pallas_call(
        _stage_c_body,
        grid=(B // Bb,),
        in_specs=[
            pl.BlockSpec((Bb, IG * 128), lambda i: (i, 0)),
            pl.BlockSpec((Bb, IG), lambda i: (i, 0)),
            pl.BlockSpec((Bb, IG), lambda i: (i, 0)),
            pl.BlockSpec((Bb, H), lambda i: (i, 0)),
            full((IG, 4 * IG)),
            full((4 * IG, IG * 128)),
            full((IG * 128, H)), full((1, H)),
            full((H, H)),
        ],
        out_specs=pl.BlockSpec((Bb, H), lambda i: (i, 0)),
        out_shape=jax.ShapeDtypeStruct((B, H), f32),
        compiler_params=pltpu.CompilerParams(
            dimension_semantics=("parallel",)),
        interpret=interpret,
    )(X2, w16, lidx, pre, G, R2, Wig2048, row(big), Wa2)


@jax.jit
def kernel(x, Ws, bs, Wo, bo, Ww, bw, v, Wig, big, Wa, ba):
    B = x.shape[0]
    f32 = jnp.float32

    si = x[:, :SELF]
    xo = jnp.pad(x[:, SELF:].reshape(B, N, C),
                 ((0, 0), (0, NP - N), (0, CP - C)))
    Wo_p = jnp.pad(Wo, ((0, CP - C), (0, 0)))
    Wig_p = jnp.pad(Wig.reshape(IG, C, H),
                    ((0, 0), (0, CP - C), (0, 0)))
    Wig2048 = jnp.broadcast_to(Wig_p[:, None], (IG, 4, CP, H)).reshape(
        IG * 4 * CP, H)
    Wwo, Wws = Ww[:H], Ww[H:]
    Wa1, Wa2, Wa3 = Wa[:H], Wa[H:2 * H], Wa[2 * H:]
    G = jnp.kron(jnp.eye(IG, dtype=f32), jnp.ones((1, 4), f32))
    R2 = jnp.kron(jnp.eye(4 * IG, dtype=f32), jnp.ones((1, CP), f32))

    w, pre, w16, lidx, g16 = _stage_a(si, xo, Ws, bs, Wo_p, bo, Wwo, Wws, bw,
                                      v, Wa1, Wa3, ba, Bb=128)
    X = _sc_gather(xo.reshape(B * NP * CP // 128, 128),
                   g16.reshape(1, B * IG))
    obs = _stage_c(X.reshape(B, IG * 128), w16, lidx, pre, G, R2, Wig2048,
                   big, Wa2, Bb=512)
    return obs, w
